# baseline (speedup 1.0000x reference)
import jax
import jax.numpy as jnp
from jax.experimental import pallas as pl
from jax.experimental.pallas import tpu as pltpu

_N, _D, _K = 8192, 64, 1024
_NB = 4096
_W = 2 * _D + 8


def _fused_kernel(sig_ref, probs_ref, h3_ref,
                  protos_ref, dist_ref, lam_ref, acc_ref, ex_ref):
    i = pl.program_id(0)
    nsteps = pl.num_programs(0)

    h3 = h3_ref[...]
    h2 = jnp.concatenate([h3[:, 0, :], h3[:, 1, :]], axis=1)

    @pl.when(i == 0)
    def _init():
        acc_ref[...] = jnp.zeros_like(acc_ref)
        ex_ref[...] = h2[0:1, :].T

    pb = probs_ref[...].astype(jnp.bfloat16)
    hb = jnp.concatenate(
        [h2, jnp.ones((h2.shape[0], 8), jnp.float32)],
        axis=1).astype(jnp.bfloat16)
    acc_ref[...] += jax.lax.dot_general(
        hb, pb, dimension_numbers=(((0,), (0,)), ((), ())),
        preferred_element_type=jnp.float32)

    @pl.when(i == nsteps - 1)
    def _epilogue():
        acc = acc_ref[...]
        psum = acc[2 * _D:2 * _D + 1, :]
        denom = jnp.where(psum == 0.0, 1.0, psum)
        protos_t = acc[:2 * _D, :] / denom
        pr_t = protos_t[:_D, :]
        pi_t = protos_t[_D:, :]
        protos_ref[0] = pr_t
        protos_ref[1] = pi_t
        mr = jnp.mean(pr_t, axis=1, keepdims=True)
        mi = jnp.mean(pi_t, axis=1, keepdims=True)
        rho = jnp.mean((pr_t - mr) ** 2 + (pi_t - mi) ** 2)
        sigma = jnp.exp(sig_ref[0])
        lam = jnp.abs(-2.0 * sigma * jnp.log(0.01)
                      + sigma * jnp.log(1.0 + rho / sigma))
        lam_ref[0] = lam
        ex = ex_ref[...]
        dist_ref[...] = jnp.sum((protos_t - ex) ** 2, axis=0, keepdims=True)


@jax.jit
def kernel(h, probs, log_sigma_l):
    k, d = _K, _D
    h3 = h.reshape(_N, 2, d)
    probs2 = probs.reshape(_N, k)
    grid = (_N // _NB,)
    protos_t2, dist, lam = pl.pallas_call(
        _fused_kernel,
        grid=grid,
        in_specs=[
            pl.BlockSpec(memory_space=pltpu.SMEM),
            pl.BlockSpec((_NB, k), lambda i: (i, 0)),
            pl.BlockSpec((_NB, 2, d), lambda i: (i, 0, 0)),
        ],
        out_specs=[
            pl.BlockSpec((2, d, k), lambda i: (0, 0, 0)),
            pl.BlockSpec((1, k), lambda i: (0, 0)),
            pl.BlockSpec(memory_space=pltpu.SMEM),
        ],
        out_shape=[
            jax.ShapeDtypeStruct((2, d, k), jnp.float32),
            jax.ShapeDtypeStruct((1, k), jnp.float32),
            jax.ShapeDtypeStruct((1,), jnp.float32),
        ],
        scratch_shapes=[pltpu.VMEM((_W, k), jnp.float32),
                        pltpu.VMEM((2 * d, 1), jnp.float32)],
        compiler_params=pltpu.CompilerParams(
            dimension_semantics=("arbitrary",)),
    )(log_sigma_l, probs2, h3)

    protos = protos_t2.transpose(0, 2, 1)[None]
    lamda = lam.reshape(())
    return (protos, dist, lamda)


# speedup vs baseline: 1.6260x; 1.6260x over previous
import jax
import jax.numpy as jnp
from jax.experimental import pallas as pl
from jax.experimental.pallas import tpu as pltpu

_N, _D, _K = 8192, 64, 1024
_NB = 2048
_W = 2 * _D + 8


def _fused_kernel(sig_ref, probs_ref, ht_ref,
                  protos_ref, dist_ref, lam_ref, acc_ref, ex_ref):
    i = pl.program_id(0)
    nsteps = pl.num_programs(0)

    ht = ht_ref[...]

    @pl.when(i == 0)
    def _init():
        acc_ref[...] = jnp.zeros_like(acc_ref)
        ex_ref[...] = ht[:, 0:1].astype(jnp.float32)

    pb = probs_ref[...].astype(jnp.bfloat16)
    hb = jnp.concatenate(
        [ht, jnp.ones((8, ht.shape[1]), jnp.bfloat16)],
        axis=0)
    acc_ref[...] += jax.lax.dot_general(
        hb, pb, dimension_numbers=(((1,), (0,)), ((), ())),
        preferred_element_type=jnp.float32)

    @pl.when(i == nsteps - 1)
    def _epilogue():
        acc = acc_ref[...]
        psum = acc[2 * _D:2 * _D + 1, :]
        denom = jnp.where(psum == 0.0, 1.0, psum)
        protos_t = acc[:2 * _D, :] / denom
        pr_t = protos_t[:_D, :]
        pi_t = protos_t[_D:, :]
        protos_ref[0] = pr_t
        protos_ref[1] = pi_t
        mr = jnp.mean(pr_t, axis=1, keepdims=True)
        mi = jnp.mean(pi_t, axis=1, keepdims=True)
        rho = jnp.mean((pr_t - mr) ** 2 + (pi_t - mi) ** 2)
        sigma = jnp.exp(sig_ref[0])
        lam = jnp.abs(-2.0 * sigma * jnp.log(0.01)
                      + sigma * jnp.log(1.0 + rho / sigma))
        lam_ref[0] = lam
        ex = ex_ref[...]
        dist_ref[...] = jnp.sum((protos_t - ex) ** 2, axis=0, keepdims=True)


@jax.jit
def kernel(h, probs, log_sigma_l):
    k, d = _K, _D
    ht = h.reshape(_N, 2 * d).astype(jnp.bfloat16).T
    probs2 = probs.reshape(_N, k)
    grid = (_N // _NB,)
    protos_t2, dist, lam = pl.pallas_call(
        _fused_kernel,
        grid=grid,
        in_specs=[
            pl.BlockSpec(memory_space=pltpu.SMEM),
            pl.BlockSpec((_NB, k), lambda i: (i, 0)),
            pl.BlockSpec((2 * d, _NB), lambda i: (0, i)),
        ],
        out_specs=[
            pl.BlockSpec((2, d, k), lambda i: (0, 0, 0)),
            pl.BlockSpec((1, k), lambda i: (0, 0)),
            pl.BlockSpec(memory_space=pltpu.SMEM),
        ],
        out_shape=[
            jax.ShapeDtypeStruct((2, d, k), jnp.float32),
            jax.ShapeDtypeStruct((1, k), jnp.float32),
            jax.ShapeDtypeStruct((1,), jnp.float32),
        ],
        scratch_shapes=[pltpu.VMEM((_W, k), jnp.float32),
                        pltpu.VMEM((2 * d, 1), jnp.float32)],
        compiler_params=pltpu.CompilerParams(
            dimension_semantics=("arbitrary",)),
    )(log_sigma_l, probs2, ht)

    protos = protos_t2.transpose(0, 2, 1)[None]
    lamda = lam.reshape(())
    return (protos, dist, lamda)
